# baseline (speedup 1.0000x reference)
import jax
import jax.numpy as jnp
from jax.experimental import pallas as pl
from jax.experimental.pallas import tpu as pltpu

_NSEM = 16


def kernel(r_idx, kv):
    b, p2, w2, c_kv = kv.shape
    topk = r_idx.shape[2]
    total = b * p2 * topk
    pq = p2 * topk

    flat_idx = r_idx.reshape(total).astype(jnp.int32)

    def body(idx_ref, kv_hbm, out_hbm, sems):
        def get_copy(i):
            bi = i // pq
            rest = i % pq
            qi = rest // topk
            ki = rest % topk
            src = idx_ref[i]
            return pltpu.make_async_copy(
                kv_hbm.at[bi, src],
                out_hbm.at[bi, qi, ki],
                sems.at[i % _NSEM],
            )

        def loop(i, carry):
            @pl.when(i >= _NSEM)
            def _():
                get_copy(i - _NSEM).wait()

            get_copy(i).start()
            return carry

        jax.lax.fori_loop(0, total, loop, 0)

        def drain(i, carry):
            get_copy(total - _NSEM + i).wait()
            return carry

        jax.lax.fori_loop(0, _NSEM, drain, 0)

    grid_spec = pltpu.PrefetchScalarGridSpec(
        num_scalar_prefetch=1,
        grid=(1,),
        in_specs=[pl.BlockSpec(memory_space=pl.ANY)],
        out_specs=pl.BlockSpec(memory_space=pl.ANY),
        scratch_shapes=[pltpu.SemaphoreType.DMA((_NSEM,))],
    )

    return pl.pallas_call(
        body,
        grid_spec=grid_spec,
        out_shape=jax.ShapeDtypeStruct((b, p2, topk, w2, c_kv), kv.dtype),
    )(flat_idx, kv)


# speedup vs baseline: 20.9594x; 20.9594x over previous
import jax
import jax.numpy as jnp
from jax.experimental import pallas as pl
from jax.experimental.pallas import tpu as pltpu


def kernel(r_idx, kv):
    b, p2, w2, c_kv = kv.shape
    topk = r_idx.shape[2]
    qk = p2 * topk
    blk = w2 * c_kv
    sub = blk // 128

    kv_r = kv.reshape(b, p2, sub, 128)
    flat_idx = r_idx.reshape(b, qk).astype(jnp.int32)

    def body(idx_ref, kv_ref, out_ref):
        bi = pl.program_id(0)
        for j in range(qk):
            out_ref[0, j] = kv_ref[0, idx_ref[bi, j]]

    grid_spec = pltpu.PrefetchScalarGridSpec(
        num_scalar_prefetch=1,
        grid=(b,),
        in_specs=[
            pl.BlockSpec((1, p2, sub, 128), lambda bi, idx_ref: (bi, 0, 0, 0))
        ],
        out_specs=pl.BlockSpec(
            (1, qk, sub, 128), lambda bi, idx_ref: (bi, 0, 0, 0)
        ),
    )

    out = pl.pallas_call(
        body,
        grid_spec=grid_spec,
        out_shape=jax.ShapeDtypeStruct((b, qk, sub, 128), kv.dtype),
    )(flat_idx, kv_r)

    return out.reshape(b, p2, topk, w2, c_kv)
